# Initial kernel scaffold; baseline (speedup 1.0000x reference)
import jax
import jax.numpy as jnp
from jax.experimental import pallas as pl


def kernel(x, W):
    raise NotImplementedError("write your pallas kernel here")


# speedup vs baseline: 1.0310x; 1.0310x over previous
import jax
import jax.numpy as jnp
from jax.experimental import pallas as pl

HIDDEN = 4096
NUM_EXPERTS = 64
TOP_K = 8
BT = 512


def _router_block(x_ref, w_ref, idx_ref, gate_ref):
    logits = jax.lax.dot_general(
        x_ref[...], w_ref[...],
        dimension_numbers=(((1,), (1,)), ((), ())),
        preferred_element_type=jnp.float32,
    )
    iota = jax.lax.broadcasted_iota(jnp.int32, logits.shape, 1)
    vals = logits
    top_vals, top_idx = [], []
    for _ in range(TOP_K):
        m = jnp.max(vals, axis=1, keepdims=True)
        idx = jnp.min(jnp.where(vals == m, iota, NUM_EXPERTS), axis=1,
                      keepdims=True)
        top_vals.append(m)
        top_idx.append(idx)
        vals = jnp.where(iota == idx, -jnp.inf, vals)
    tv = jnp.concatenate(top_vals, axis=1)
    ti = jnp.concatenate(top_idx, axis=1)
    e = jnp.exp(tv - tv[:, 0:1])
    gate_ref[...] = e / jnp.sum(e, axis=1, keepdims=True)
    idx_ref[...] = ti


def kernel(x, W):
    tokens = x.shape[0]
    grid = (tokens // BT,)
    return pl.pallas_call(
        _router_block,
        grid=grid,
        in_specs=[
            pl.BlockSpec((BT, HIDDEN), lambda i: (i, 0)),
            pl.BlockSpec((NUM_EXPERTS, HIDDEN), lambda i: (0, 0)),
        ],
        out_specs=[
            pl.BlockSpec((BT, TOP_K), lambda i: (i, 0)),
            pl.BlockSpec((BT, TOP_K), lambda i: (i, 0)),
        ],
        out_shape=[
            jax.ShapeDtypeStruct((tokens, TOP_K), jnp.int32),
            jax.ShapeDtypeStruct((tokens, TOP_K), jnp.float32),
        ],
    )(x, W)


# speedup vs baseline: 1.7404x; 1.6880x over previous
import jax
import jax.numpy as jnp
from jax.experimental import pallas as pl

HIDDEN = 4096
NUM_EXPERTS = 64
TOP_K = 8
BT = 512


def _router_block(x_ref, w_ref, idx_ref, gate_ref):
    logits = jax.lax.dot_general(
        x_ref[...], w_ref[...],
        dimension_numbers=(((1,), (1,)), ((), ())),
        preferred_element_type=jnp.float32,
    )
    lt = logits.T
    iota = jax.lax.broadcasted_iota(jnp.int32, lt.shape, 0)
    vals = lt
    top_vals, top_idx = [], []
    for _ in range(TOP_K):
        m = jnp.max(vals, axis=0, keepdims=True)
        idx = jnp.min(jnp.where(vals == m, iota, NUM_EXPERTS), axis=0,
                      keepdims=True)
        top_vals.append(m)
        top_idx.append(idx)
        vals = jnp.where(iota == idx, -jnp.inf, vals)
    tv = jnp.concatenate(top_vals, axis=0)
    ti = jnp.concatenate(top_idx, axis=0)
    e = jnp.exp(tv - tv[0:1, :])
    gate_ref[...] = e / jnp.sum(e, axis=0, keepdims=True)
    idx_ref[...] = ti


def kernel(x, W):
    tokens = x.shape[0]
    grid = (tokens // BT,)
    idx_t, gates_t = pl.pallas_call(
        _router_block,
        grid=grid,
        in_specs=[
            pl.BlockSpec((BT, HIDDEN), lambda i: (i, 0)),
            pl.BlockSpec((NUM_EXPERTS, HIDDEN), lambda i: (0, 0)),
        ],
        out_specs=[
            pl.BlockSpec((TOP_K, BT), lambda i: (0, i)),
            pl.BlockSpec((TOP_K, BT), lambda i: (0, i)),
        ],
        out_shape=[
            jax.ShapeDtypeStruct((TOP_K, tokens), jnp.int32),
            jax.ShapeDtypeStruct((TOP_K, tokens), jnp.float32),
        ],
    )(x, W)
    return idx_t.T, gates_t.T


# speedup vs baseline: 1.8315x; 1.0523x over previous
import jax
import jax.numpy as jnp
from jax.experimental import pallas as pl

HIDDEN = 4096
NUM_EXPERTS = 64
TOP_K = 8
BT = 1024


def _router_block(x_ref, w_ref, idx_ref, gate_ref):
    logits = jax.lax.dot_general(
        x_ref[...], w_ref[...],
        dimension_numbers=(((1,), (1,)), ((), ())),
        preferred_element_type=jnp.float32,
    )
    lt = logits.T
    iota = jax.lax.broadcasted_iota(jnp.int32, lt.shape, 0)
    vals = lt
    top_vals, top_idx = [], []
    for _ in range(TOP_K):
        m = jnp.max(vals, axis=0, keepdims=True)
        idx = jnp.min(jnp.where(vals == m, iota, NUM_EXPERTS), axis=0,
                      keepdims=True)
        top_vals.append(m)
        top_idx.append(idx)
        vals = jnp.where(iota == idx, -jnp.inf, vals)
    tv = jnp.concatenate(top_vals, axis=0)
    ti = jnp.concatenate(top_idx, axis=0)
    e = jnp.exp(tv - tv[0:1, :])
    gate_ref[...] = e / jnp.sum(e, axis=0, keepdims=True)
    idx_ref[...] = ti


def kernel(x, W):
    tokens = x.shape[0]
    grid = (tokens // BT,)
    idx_t, gates_t = pl.pallas_call(
        _router_block,
        grid=grid,
        in_specs=[
            pl.BlockSpec((BT, HIDDEN), lambda i: (i, 0)),
            pl.BlockSpec((NUM_EXPERTS, HIDDEN), lambda i: (0, 0)),
        ],
        out_specs=[
            pl.BlockSpec((TOP_K, BT), lambda i: (0, i)),
            pl.BlockSpec((TOP_K, BT), lambda i: (0, i)),
        ],
        out_shape=[
            jax.ShapeDtypeStruct((TOP_K, tokens), jnp.int32),
            jax.ShapeDtypeStruct((TOP_K, tokens), jnp.float32),
        ],
    )(x, W)
    return idx_t.T, gates_t.T
